# baseline (speedup 1.0000x reference)
import jax
import jax.numpy as jnp
from jax import lax
from jax.experimental import pallas as pl
from jax.experimental.pallas import tpu as pltpu
from jax.experimental.pallas import tpu_sc as plsc

_T, _E, _K = 16384, 64, 8
_NC, _NS, _L = 2, 16, 16
_NW = _NC * _NS
_TPW = _T // _NW
_H = _TPW // 2
_GH = _H // _L

_SORT8 = ((0, 1), (2, 3), (4, 5), (6, 7),
          (0, 2), (1, 3), (4, 6), (5, 7),
          (1, 2), (5, 6),
          (0, 4), (1, 5), (2, 6), (3, 7),
          (2, 4), (3, 5),
          (1, 2), (3, 4), (5, 6))
_MERGE = ((0, 4), (1, 5), (2, 6), (3, 7),
          (0, 2), (1, 3), (4, 6), (5, 7),
          (0, 1), (2, 3), (4, 5), (6, 7))
_REPAIR = ((0, 1), (2, 3), (4, 5), (6, 7),
           (1, 2), (3, 4), (5, 6),
           (0, 1), (2, 3), (4, 5), (6, 7))


def _ce(a, i, j):
    hi = jnp.maximum(a[i], a[j])
    lo = jnp.minimum(a[i], a[j])
    a[i] = hi
    a[j] = lo


def _topk_body(logits_hbm, w_hbm, id_hbm, in_v, w_v, id_v):
    wid = lax.axis_index("s") * _NC + lax.axis_index("c")
    base = wid * _TPW
    lanes = lax.iota(jnp.int32, _L)

    for half in range(2):
        hbase = base + half * _H
        pltpu.sync_copy(logits_hbm.at[pl.ds(hbase, _H)], in_v)

        def body(g, carry):
            rows = g * _L + lanes
            t = None
            for c in range(_E // 8):
                d = []
                for j in range(8):
                    e = c * 8 + j
                    col = jnp.full((_L,), e, jnp.int32)
                    v = plsc.load_gather(in_v, [rows, col])
                    x = plsc.bitcast(v, jnp.int32)
                    d.append(plsc.bitcast(
                        (x & jnp.int32(-64)) | jnp.int32(63 - e),
                        jnp.float32))
                for i, j in _SORT8:
                    _ce(d, i, j)
                if t is None:
                    t = d
                else:
                    mrg = [jnp.maximum(t[i], d[7 - i]) for i in range(8)]
                    for i, j in _MERGE:
                        _ce(mrg, i, j)
                    t = mrg
            ids = [jnp.int32(63) -
                   (plsc.bitcast(tk, jnp.int32) & jnp.int32(63))
                   for tk in t]
            vals = [plsc.load_gather(in_v, [rows, ids[j]])
                    for j in range(_K)]
            for i, j in _REPAIR:
                gt = vals[i] >= vals[j]
                vhi = jnp.where(gt, vals[i], vals[j])
                vlo = jnp.where(gt, vals[j], vals[i])
                ihi = jnp.where(gt, ids[i], ids[j])
                ilo = jnp.where(gt, ids[j], ids[i])
                vals[i], vals[j] = vhi, vlo
                ids[i], ids[j] = ihi, ilo
            ws = [jnp.exp(vals[j] - vals[0]) for j in range(_K)]
            s = ws[0]
            for j in range(1, _K):
                s = s + ws[j]
            r = jnp.float32(1.0) / s
            for j in range(_K):
                col = jnp.full((_L,), j, jnp.int32)
                plsc.store_scatter(w_v, [rows, col], ws[j] * r)
                plsc.store_scatter(id_v, [rows, col], ids[j])
            return carry

        lax.fori_loop(0, _GH, body, None)
        pltpu.sync_copy(w_v, w_hbm.at[pl.ds(hbase, _H)])
        pltpu.sync_copy(id_v, id_hbm.at[pl.ds(hbase, _H)])


@jax.jit
def _run(logits):
    mesh = plsc.VectorSubcoreMesh(core_axis_name="c", subcore_axis_name="s")
    return pl.kernel(
        _topk_body,
        out_type=[jax.ShapeDtypeStruct((_T, _K), jnp.float32),
                  jax.ShapeDtypeStruct((_T, _K), jnp.int32)],
        mesh=mesh,
        scratch_types=[pltpu.VMEM((_H, _E), jnp.float32),
                       pltpu.VMEM((_H, _K), jnp.float32),
                       pltpu.VMEM((_H, _K), jnp.int32)],
        compiler_params=pltpu.CompilerParams(needs_layout_passes=False),
    )(logits)


def kernel(router_logits_fp32, topk_ids, topk_weights):
    w, ids = _run(router_logits_fp32)
    return w, ids.astype(jnp.int64)


# speedup vs baseline: 1.0758x; 1.0758x over previous
import jax
import jax.numpy as jnp
from jax import lax
from jax.experimental import pallas as pl
from jax.experimental.pallas import tpu as pltpu
from jax.experimental.pallas import tpu_sc as plsc

_T, _E, _K = 16384, 64, 8
_NC, _NS, _L = 2, 16, 16
_NW = _NC * _NS
_TPW = _T // _NW
_H = _TPW // 2
_GH = _H // _L

_SORT8 = ((0, 1), (2, 3), (4, 5), (6, 7),
          (0, 2), (1, 3), (4, 6), (5, 7),
          (1, 2), (5, 6),
          (0, 4), (1, 5), (2, 6), (3, 7),
          (2, 4), (3, 5),
          (1, 2), (3, 4), (5, 6))
_MERGE = ((0, 4), (1, 5), (2, 6), (3, 7),
          (0, 2), (1, 3), (4, 6), (5, 7),
          (0, 1), (2, 3), (4, 5), (6, 7))
_REPAIR = ((0, 1), (2, 3), (4, 5), (6, 7),
           (1, 2), (3, 4), (5, 6),
           (0, 1), (2, 3), (4, 5), (6, 7))


def _ce(a, i, j):
    hi = jnp.maximum(a[i], a[j])
    lo = jnp.minimum(a[i], a[j])
    a[i] = hi
    a[j] = lo


_SKEW = _E + 1


def _topk_body(logits_hbm, w_hbm, id_hbm, in_v, skew_v, w_v, id_v):
    wid = lax.axis_index("s") * _NC + lax.axis_index("c")
    base = wid * _TPW
    lanes = lax.iota(jnp.int32, _L)

    for half in range(2):
        hbase = base + half * _H
        pltpu.sync_copy(logits_hbm.at[pl.ds(hbase, _H)], in_v)

        def skew(tok, carry):
            for c in range(_E // _L):
                skew_v[pl.ds(tok * _SKEW + c * _L, _L)] = (
                    in_v[tok, pl.ds(c * _L, _L)])
            return carry

        lax.fori_loop(0, _H, skew, None)

        def body(g, carry):
            rows = g * _L + lanes
            srows = rows * _SKEW
            t = None
            for c in range(_E // 8):
                d = []
                for j in range(8):
                    e = c * 8 + j
                    v = plsc.load_gather(skew_v, [srows + e])
                    x = plsc.bitcast(v, jnp.int32)
                    d.append(plsc.bitcast(
                        (x & jnp.int32(-64)) | jnp.int32(63 - e),
                        jnp.float32))
                for i, j in _SORT8:
                    _ce(d, i, j)
                if t is None:
                    t = d
                else:
                    mrg = [jnp.maximum(t[i], d[7 - i]) for i in range(8)]
                    for i, j in _MERGE:
                        _ce(mrg, i, j)
                    t = mrg
            ids = [jnp.int32(63) -
                   (plsc.bitcast(tk, jnp.int32) & jnp.int32(63))
                   for tk in t]
            vals = [plsc.load_gather(skew_v, [srows + ids[j]])
                    for j in range(_K)]
            for i, j in _REPAIR:
                gt = vals[i] >= vals[j]
                vhi = jnp.where(gt, vals[i], vals[j])
                vlo = jnp.where(gt, vals[j], vals[i])
                ihi = jnp.where(gt, ids[i], ids[j])
                ilo = jnp.where(gt, ids[j], ids[i])
                vals[i], vals[j] = vhi, vlo
                ids[i], ids[j] = ihi, ilo
            ws = [jnp.exp(vals[j] - vals[0]) for j in range(_K)]
            s = ws[0]
            for j in range(1, _K):
                s = s + ws[j]
            r = jnp.float32(1.0) / s
            for j in range(_K):
                col = jnp.full((_L,), j, jnp.int32)
                plsc.store_scatter(w_v, [rows, col], ws[j] * r)
                plsc.store_scatter(id_v, [rows, col], ids[j])
            return carry

        lax.fori_loop(0, _GH, body, None)
        pltpu.sync_copy(w_v, w_hbm.at[pl.ds(hbase, _H)])
        pltpu.sync_copy(id_v, id_hbm.at[pl.ds(hbase, _H)])


@jax.jit
def _run(logits):
    mesh = plsc.VectorSubcoreMesh(core_axis_name="c", subcore_axis_name="s")
    return pl.kernel(
        _topk_body,
        out_type=[jax.ShapeDtypeStruct((_T, _K), jnp.float32),
                  jax.ShapeDtypeStruct((_T, _K), jnp.int32)],
        mesh=mesh,
        scratch_types=[pltpu.VMEM((_H, _E), jnp.float32),
                       pltpu.VMEM((_H * _SKEW,), jnp.float32),
                       pltpu.VMEM((_H, _K), jnp.float32),
                       pltpu.VMEM((_H, _K), jnp.int32)],
        compiler_params=pltpu.CompilerParams(needs_layout_passes=False),
    )(logits)


def kernel(router_logits_fp32, topk_ids, topk_weights):
    w, ids = _run(router_logits_fp32)
    return w, ids.astype(jnp.int64)


# speedup vs baseline: 1.6177x; 1.5036x over previous
import jax
import jax.numpy as jnp
from jax import lax
from jax.experimental import pallas as pl
from jax.experimental.pallas import tpu as pltpu
from jax.experimental.pallas import tpu_sc as plsc

_T, _E, _K = 16384, 64, 8
_NC, _NS, _L = 2, 16, 16
_NW = _NC * _NS
_TPW = _T // _NW
_G = _TPW // _L
_SKEW = _E + 1

_SORT8 = ((0, 1), (2, 3), (4, 5), (6, 7),
          (0, 2), (1, 3), (4, 6), (5, 7),
          (1, 2), (5, 6),
          (0, 4), (1, 5), (2, 6), (3, 7),
          (2, 4), (3, 5),
          (1, 2), (3, 4), (5, 6))
_MERGE = ((0, 4), (1, 5), (2, 6), (3, 7),
          (0, 2), (1, 3), (4, 6), (5, 7),
          (0, 1), (2, 3), (4, 5), (6, 7))
_REPAIR = ((0, 1), (2, 3), (4, 5), (6, 7),
           (1, 2), (3, 4), (5, 6),
           (0, 1), (2, 3), (4, 5), (6, 7))


def _ce(a, i, j):
    hi = jnp.maximum(a[i], a[j])
    lo = jnp.minimum(a[i], a[j])
    a[i] = hi
    a[j] = lo


def _topk_body(logits_hbm, w_hbm, id_hbm, in_v, skew_v, w_v, id_v):
    wid = lax.axis_index("s") * _NC + lax.axis_index("c")
    base = wid * _TPW
    lanes = lax.iota(jnp.int32, _L)

    pltpu.sync_copy(logits_hbm.at[pl.ds(base, _TPW)], in_v)

    def skew(tok, carry):
        for c in range(_E // _L):
            skew_v[pl.ds(tok * _SKEW + c * _L, _L)] = (
                in_v[tok, pl.ds(c * _L, _L)])
        return carry

    lax.fori_loop(0, _TPW, skew, None)

    def body(g, carry):
        rows = g * _L + lanes
        srows = rows * _SKEW
        t = None
        for c in range(_E // 8):
            d = []
            for j in range(8):
                e = c * 8 + j
                v = plsc.load_gather(skew_v, [srows + e])
                x = plsc.bitcast(v, jnp.int32)
                d.append(plsc.bitcast(
                    (x & jnp.int32(-64)) | jnp.int32(63 - e),
                    jnp.float32))
            for i, j in _SORT8:
                _ce(d, i, j)
            if t is None:
                t = d
            else:
                mrg = [jnp.maximum(t[i], d[7 - i]) for i in range(8)]
                for i, j in _MERGE:
                    _ce(mrg, i, j)
                t = mrg
        ids = [jnp.int32(63) -
               (plsc.bitcast(tk, jnp.int32) & jnp.int32(63))
               for tk in t]
        vals = [plsc.load_gather(skew_v, [srows + ids[j]])
                for j in range(_K)]
        for i, j in _REPAIR:
            gt = vals[i] >= vals[j]
            vhi = jnp.where(gt, vals[i], vals[j])
            vlo = jnp.where(gt, vals[j], vals[i])
            ihi = jnp.where(gt, ids[i], ids[j])
            ilo = jnp.where(gt, ids[j], ids[i])
            vals[i], vals[j] = vhi, vlo
            ids[i], ids[j] = ihi, ilo
        ws = [jnp.exp(vals[j] - vals[0]) for j in range(_K)]
        s = ws[0]
        for j in range(1, _K):
            s = s + ws[j]
        r = jnp.float32(1.0) / s
        for j in range(_K):
            w_v[j, pl.ds(g * _L, _L)] = ws[j] * r
            id_v[j, pl.ds(g * _L, _L)] = ids[j]
        return carry

    lax.fori_loop(0, _G, body, None)
    pltpu.sync_copy(w_v, w_hbm.at[:, pl.ds(base, _TPW)])
    pltpu.sync_copy(id_v, id_hbm.at[:, pl.ds(base, _TPW)])


@jax.jit
def _run(logits):
    mesh = plsc.VectorSubcoreMesh(core_axis_name="c", subcore_axis_name="s")
    return pl.kernel(
        _topk_body,
        out_type=[jax.ShapeDtypeStruct((_K, _T), jnp.float32),
                  jax.ShapeDtypeStruct((_K, _T), jnp.int32)],
        mesh=mesh,
        scratch_types=[pltpu.VMEM((_TPW, _E), jnp.float32),
                       pltpu.VMEM((_TPW * _SKEW,), jnp.float32),
                       pltpu.VMEM((_K, _TPW), jnp.float32),
                       pltpu.VMEM((_K, _TPW), jnp.int32)],
        compiler_params=pltpu.CompilerParams(needs_layout_passes=False),
    )(logits)


def kernel(router_logits_fp32, topk_ids, topk_weights):
    w, ids = _run(router_logits_fp32)
    return w.T, ids.T.astype(jnp.int64)
